# baseline (speedup 1.0000x reference)
import jax
import jax.numpy as jnp
from jax import lax
from jax.experimental import pallas as pl
from jax.experimental.pallas import tpu as pltpu
from jax.experimental.pallas import tpu_sc as plsc

UNITS = 128
CHUNK = 128
NBUF = 4
LOOKAHEAD = 2


def _sc_body(table_hbm, idx_hbm, pos_hbm, out_hbm, table_sp, idx_all, pos_all,
             rows0, rows1, rows2, rows3,
             sin0, sin1, sin2, sin3, sout0, sout1, sout2, sout3):
    rows = (rows0, rows1, rows2, rows3)
    sin = (sin0, sin1, sin2, sin3)
    sout = (sout0, sout1, sout2, sout3)

    info = plsc.get_sparse_core_info()
    nc, ns = info.num_cores, info.num_subcores
    nw = nc * ns
    cid = lax.axis_index("c")
    sid = lax.axis_index("s")
    wid = sid * nc + cid

    @pl.when(sid == 0)
    def _():
        pltpu.sync_copy(table_hbm, table_sp)

    plsc.subcore_barrier()

    n_rows = out_hbm.shape[0]
    rows_per_w = n_rows // nw
    n_chunks = rows_per_w // CHUNK
    n_groups = n_chunks // NBUF
    base = wid * rows_per_w

    pltpu.sync_copy(idx_hbm.at[pl.ds(base, rows_per_w)], idx_all)
    pltpu.sync_copy(pos_hbm.at[wid], pos_all)

    def gather(c, b):
        return pltpu.async_copy(table_sp.at[idx_all.at[pl.ds(c * CHUNK, CHUNK)]],
                                rows[b], sin[b])

    def wait_gather(b):
        pltpu.make_async_copy(table_sp.at[idx_all.at[pl.ds(0, CHUNK)]],
                              rows[b], sin[b]).wait()

    def put(c, b):
        return pltpu.async_copy(rows[b], out_hbm.at[pos_all.at[c]], sout[b])

    def wait_put(b):
        pltpu.make_async_copy(rows[b], out_hbm.at[pos_all.at[0]], sout[b]).wait()

    for c in range(LOOKAHEAD):
        gather(c, c % NBUF)

    def group(g, carry):
        for db in range(NBUF):
            c = NBUF * g + db
            bg = (db + LOOKAHEAD) % NBUF

            @pl.when(jnp.logical_and(c + LOOKAHEAD < n_chunks,
                                     c + LOOKAHEAD >= NBUF))
            def _():
                wait_put(bg)

            @pl.when(c + LOOKAHEAD < n_chunks)
            def _():
                gather(c + LOOKAHEAD, bg)

            wait_gather(db)
            put(c, db)
        return carry

    lax.fori_loop(0, n_groups, group, 0)

    for b in range(NBUF):
        wait_put(b)


def kernel(inputs, w1, w2):
    B, T = inputs.shape
    U = w1.shape[1]
    n = B * T
    idx = inputs.reshape(-1).astype(jnp.int32)
    table = jnp.concatenate([jnp.zeros_like(w1), w1, w2], axis=0)
    pos = jnp.arange(n, dtype=jnp.int32).reshape(32, (n // 32) // CHUNK, CHUNK)
    mesh = plsc.VectorSubcoreMesh(core_axis_name="c", subcore_axis_name="s")
    rows_per_w = n // 32
    k = pl.kernel(
        _sc_body,
        out_type=jax.ShapeDtypeStruct((n, U), jnp.float32),
        mesh=mesh,
        scratch_types=(
            [pltpu.VMEM_SHARED((3, U), jnp.float32),
             pltpu.VMEM((rows_per_w,), jnp.int32),
             pltpu.VMEM((rows_per_w // CHUNK, CHUNK), jnp.int32)]
            + [pltpu.VMEM((CHUNK, U), jnp.float32)] * NBUF
            + [pltpu.SemaphoreType.DMA] * (2 * NBUF)
        ),
    )
    out = k(table, idx, pos)
    return out.reshape(B, T, U)


# speedup vs baseline: 1.0107x; 1.0107x over previous
import jax
import jax.numpy as jnp
from jax import lax
from jax.experimental import pallas as pl
from jax.experimental.pallas import tpu as pltpu
from jax.experimental.pallas import tpu_sc as plsc

UNITS = 128
BLK = 128
SCK = 256
NW = 32


def _sc_body(table_hbm, idx_hbm, out_hbm, table_sp,
             src0, src1, src2, cidx, offbuf, pbuf, list0, list1, list2,
             idxb0, idxb1, si0, si1, ss0, ss1, ss2):
    srcs = (src0, src1, src2)
    lists = (list0, list1, list2)
    ssem = (ss0, ss1, ss2)

    info = plsc.get_sparse_core_info()
    nc, ns = info.num_cores, info.num_subcores
    nw = nc * ns
    cid = lax.axis_index("c")
    sid = lax.axis_index("s")
    wid = sid * nc + cid

    @pl.when(sid == 0)
    def _():
        pltpu.sync_copy(table_hbm, table_sp)

    plsc.subcore_barrier()

    n_rows = out_hbm.shape[0]
    rows_per_w = n_rows // nw
    n_sck = rows_per_w // SCK
    n_pairs = n_sck // 2
    base = wid * rows_per_w

    iota = lax.iota(jnp.int32, 16)
    shift_src = [jnp.maximum(iota - k, 0) for k in (1, 2, 4, 8)]
    shift_msk = [iota >= k for k in (1, 2, 4, 8)]
    lane15 = jnp.full((16,), 15, jnp.int32)
    zvec = jnp.zeros((16,), jnp.int32)

    def prefix_incl(x):
        for s, mk in zip(shift_src, shift_msk):
            pbuf[pl.ds(0, 16)] = x
            sh = plsc.load_gather(pbuf, [s])
            x = x + jnp.where(mk, sh, zvec)
        return x

    def lane15_bcast(x):
        pbuf[pl.ds(0, 16)] = x
        return plsc.load_gather(pbuf, [lane15])

    for v in range(3):
        for k in range(8):
            cidx[pl.ds(16 * k, 16)] = jnp.full((16,), v, jnp.int32)
        pltpu.async_copy(table_sp.at[cidx], srcs[v], ssem[v]).wait()

    def prefetch(sc, buf, sem):
        return pltpu.async_copy(idx_hbm.at[pl.ds(base + sc * SCK, SCK)], buf, sem)

    def wait_prefetch(buf, sem):
        pltpu.make_async_copy(idx_hbm.at[pl.ds(base, SCK)], buf, sem).wait()

    def scatter_block(v, j):
        return pltpu.async_copy(srcs[v], out_hbm.at[lists[v].at[j]], ssem[v])

    def wait_scatter(v):
        pltpu.make_async_copy(srcs[v], out_hbm.at[lists[v].at[0]], ssem[v]).wait()

    prefetch(0, idxb0, si0)
    prefetch(1, idxb1, si1)

    zero = jnp.zeros((16,), jnp.int32)
    for v in range(3):
        offbuf[pl.ds(16 * v, 16)] = zero

    def do_superchunk(sc, buf):
        new_offs = [offbuf[pl.ds(16 * v, 16)] for v in range(3)]
        for k in range(SCK // 16):
            idx16 = buf[pl.ds(16 * k, 16)]
            pos16 = (base + sc * SCK + 16 * k) + iota
            ones = jnp.full((16,), 1, jnp.int32)
            for v in range(3):
                m = idx16 == v
                mi = jnp.where(m, ones, zvec)
                pf = prefix_incl(mi)
                slots = new_offs[v] + (pf - mi)
                plsc.store_scatter(lists[v],
                                   [slots >> 7, slots & 127], pos16, mask=m)
                new_offs[v] = new_offs[v] + lane15_bcast(pf)
        for v in range(3):
            offbuf[pl.ds(16 * v, 16)] = new_offs[v]
        return new_offs

    def flush(v, done):
        nb = offbuf[pl.ds(16 * v, 16)][0] >> 7

        def issue(j, c):
            scatter_block(v, j)
            return c

        lax.fori_loop(done, nb, issue, 0)
        return nb

    def pair(g, carry):
        dones = list(carry)
        for half in range(2):
            sc = 2 * g + half
            buf = (idxb0, idxb1)[half]
            sem = (si0, si1)[half]
            wait_prefetch(buf, sem)
            do_superchunk(sc, buf)

            @pl.when(sc + 2 < n_sck)
            def _():
                prefetch(sc + 2, buf, sem)

            for v in range(3):
                dones[v] = flush(v, dones[v])
        return tuple(dones)

    zs = jnp.zeros((), jnp.int32)
    d0, d1, d2 = lax.fori_loop(0, n_pairs, pair, (zs, zs, zs))

    for v, dv in ((0, d0), (1, d1), (2, d2)):
        cnt = offbuf[pl.ds(16 * v, 16)][0]
        rem = cnt & 127
        nb = cnt >> 7

        @pl.when(rem != 0)
        def _():
            padvec = plsc.load_gather(lists[v], [zero, zero])
            for k in range(8):
                cur = lists[v][nb, pl.ds(16 * k, 16)]
                keep = (iota + 16 * k) < rem
                lists[v][nb, pl.ds(16 * k, 16)] = jnp.where(keep, cur, padvec)
            scatter_block(v, nb)

        def drain(j, c):
            wait_scatter(v)
            return c

        n_drain = dv + jnp.where(rem != 0, 1, 0).astype(jnp.int32)
        lax.fori_loop(0, n_drain, drain, 0)


def kernel(inputs, w1, w2):
    B, T = inputs.shape
    U = w1.shape[1]
    n = B * T
    idx = inputs.reshape(-1).astype(jnp.int32)
    table = jnp.concatenate([jnp.zeros_like(w1), w1, w2], axis=0)
    mesh = plsc.VectorSubcoreMesh(core_axis_name="c", subcore_axis_name="s")
    rows_per_w = n // NW
    nl = rows_per_w // BLK
    k = pl.kernel(
        _sc_body,
        out_type=jax.ShapeDtypeStruct((n, U), jnp.float32),
        mesh=mesh,
        compiler_params=pltpu.CompilerParams(needs_layout_passes=False),
        scratch_types=(
            [pltpu.VMEM_SHARED((3, U), jnp.float32)]
            + [pltpu.VMEM((BLK, U), jnp.float32)] * 3
            + [pltpu.VMEM((BLK,), jnp.int32)]
            + [pltpu.VMEM((48,), jnp.int32)]
            + [pltpu.VMEM((16,), jnp.int32)]
            + [pltpu.VMEM((nl, BLK), jnp.int32)] * 3
            + [pltpu.VMEM((SCK,), jnp.int32)] * 2
            + [pltpu.SemaphoreType.DMA] * 5
        ),
    )
    out = k(table, idx)
    return out.reshape(B, T, U)


# speedup vs baseline: 1.2074x; 1.1947x over previous
import jax
import jax.numpy as jnp
from jax import lax
from jax.experimental import pallas as pl
from jax.experimental.pallas import tpu as pltpu
from jax.experimental.pallas import tpu_sc as plsc

UNITS = 128
BLK = 128
SCK = 256
NW = 32


def _sc_body(table_hbm, idx_hbm, out_hbm, table_sp,
             src0, src1, src2, cidx, offbuf, pbuf, powbuf, biglist,
             idxb0, idxb1, si0, si1, ss0, ss1, ss2):
    srcs = (src0, src1, src2)
    ssem = (ss0, ss1, ss2)

    info = plsc.get_sparse_core_info()
    nc, ns = info.num_cores, info.num_subcores
    nw = nc * ns
    cid = lax.axis_index("c")
    sid = lax.axis_index("s")
    wid = sid * nc + cid

    @pl.when(sid == 0)
    def _():
        pltpu.sync_copy(table_hbm, table_sp)

    plsc.subcore_barrier()

    n_rows = out_hbm.shape[0]
    rows_per_w = n_rows // nw
    n_sck = rows_per_w // SCK
    n_pairs = n_sck // 2
    nl = rows_per_w // BLK
    base = wid * rows_per_w

    iota = lax.iota(jnp.int32, 16)
    shift_src = [jnp.maximum(iota - k, 0) for k in (1, 2, 4, 8)]
    shift_msk = [iota >= k for k in (1, 2, 4, 8)]
    lane15 = jnp.full((16,), 15, jnp.int32)
    zvec = jnp.zeros((16,), jnp.int32)
    ones = jnp.full((16,), 1, jnp.int32)
    f31 = jnp.full((16,), 31, jnp.int32)
    iota5 = jnp.minimum(iota * 5, f31 - 1)

    def prefix_incl(x):
        for s, mk in zip(shift_src, shift_msk):
            pbuf[pl.ds(0, 16)] = x
            x = x + jnp.where(mk, plsc.load_gather(pbuf, [s]), zvec)
        return x

    def lane15_bcast(x):
        pbuf[pl.ds(0, 16)] = x
        return plsc.load_gather(pbuf, [lane15])

    for v in range(3):
        for k in range(8):
            cidx[pl.ds(16 * k, 16)] = jnp.full((16,), v, jnp.int32)
        pltpu.async_copy(table_sp.at[cidx], srcs[v], ssem[v]).wait()

    powbuf[pl.ds(0, 16)] = jnp.where(iota5 < f31, ones << iota5, ones)
    offbuf[pl.ds(0, 16)] = iota * rows_per_w

    def prefetch(sc, buf, sem):
        return pltpu.async_copy(idx_hbm.at[pl.ds(base + sc * SCK, SCK)], buf, sem)

    def wait_prefetch(buf, sem):
        pltpu.make_async_copy(idx_hbm.at[pl.ds(base, SCK)], buf, sem).wait()

    def scatter_block(v, j):
        return pltpu.async_copy(srcs[v], out_hbm.at[biglist.at[v * nl + j]],
                                ssem[v])

    def wait_scatter(v):
        pltpu.make_async_copy(srcs[v], out_hbm.at[biglist.at[0]], ssem[v]).wait()

    prefetch(0, idxb0, si0)
    prefetch(1, idxb1, si1)

    def do_superchunk(sc, buf):
        off_all = offbuf[pl.ds(0, 16)]
        for k in range(SCK // 16):
            idx16 = buf[pl.ds(16 * k, 16)]
            pos16 = (base + sc * SCK + 16 * k) + iota
            enc = plsc.load_gather(powbuf, [idx16])
            pf = prefix_incl(enc)
            rank = ((pf >> (idx16 * 5)) & f31) - ones
            offsel = plsc.load_gather(offbuf, [idx16])
            slot = offsel + rank
            plsc.store_scatter(biglist, [slot >> 7, slot & 127], pos16)
            tot = lane15_bcast(pf)
            off_all = off_all + ((tot >> iota5) & f31)
            offbuf[pl.ds(0, 16)] = off_all

    def flush(v, done):
        nb = (offbuf[pl.ds(0, 16)][v] - v * rows_per_w) >> 7

        def issue(j, c):
            scatter_block(v, j)
            return c

        lax.fori_loop(done, nb, issue, 0)
        return nb

    def pair(g, carry):
        dones = list(carry)
        for half in range(2):
            sc = 2 * g + half
            buf = (idxb0, idxb1)[half]
            sem = (si0, si1)[half]
            wait_prefetch(buf, sem)
            do_superchunk(sc, buf)

            @pl.when(sc + 2 < n_sck)
            def _():
                prefetch(sc + 2, buf, sem)

            for v in range(3):
                dones[v] = flush(v, dones[v])
        return tuple(dones)

    zs = jnp.zeros((), jnp.int32)
    d0, d1, d2 = lax.fori_loop(0, n_pairs, pair, (zs, zs, zs))

    for v, dv in ((0, d0), (1, d1), (2, d2)):
        cnt = offbuf[pl.ds(0, 16)][v] - v * rows_per_w
        rem = cnt & 127
        nb = cnt >> 7
        gr = v * nl + nb

        @pl.when(rem != 0)
        def _():
            padvec = plsc.load_gather(biglist,
                                      [jnp.full((16,), v * nl, jnp.int32), zvec])
            for k in range(8):
                cur = biglist[gr, pl.ds(16 * k, 16)]
                keep = (iota + 16 * k) < rem
                biglist[gr, pl.ds(16 * k, 16)] = jnp.where(keep, cur, padvec)
            scatter_block(v, nb)

        def drain(j, c):
            wait_scatter(v)
            return c

        n_drain = dv + jnp.where(rem != 0, 1, 0).astype(jnp.int32)
        lax.fori_loop(0, n_drain, drain, 0)


def kernel(inputs, w1, w2):
    B, T = inputs.shape
    U = w1.shape[1]
    n = B * T
    idx = inputs.reshape(-1).astype(jnp.int32)
    table = jnp.concatenate([jnp.zeros_like(w1), w1, w2], axis=0)
    mesh = plsc.VectorSubcoreMesh(core_axis_name="c", subcore_axis_name="s")
    rows_per_w = n // NW
    nl = rows_per_w // BLK
    k = pl.kernel(
        _sc_body,
        out_type=jax.ShapeDtypeStruct((n, U), jnp.float32),
        mesh=mesh,
        compiler_params=pltpu.CompilerParams(needs_layout_passes=False),
        scratch_types=(
            [pltpu.VMEM_SHARED((3, U), jnp.float32)]
            + [pltpu.VMEM((BLK, U), jnp.float32)] * 3
            + [pltpu.VMEM((BLK,), jnp.int32)]
            + [pltpu.VMEM((16,), jnp.int32)]
            + [pltpu.VMEM((16,), jnp.int32)]
            + [pltpu.VMEM((16,), jnp.int32)]
            + [pltpu.VMEM((3 * nl, BLK), jnp.int32)]
            + [pltpu.VMEM((SCK,), jnp.int32)] * 2
            + [pltpu.SemaphoreType.DMA] * 5
        ),
    )
    out = k(table, idx)
    return out.reshape(B, T, U)
